# Initial kernel scaffold; baseline (speedup 1.0000x reference)
import jax
import jax.numpy as jnp
from jax.experimental import pallas as pl


def kernel(memory, raw_msg, W_lin, b_lin, W_l, b_l, W_r, b_r, n_id, edge_index):
    raise NotImplementedError("write your pallas kernel here")


# speedup vs baseline: 3.6185x; 3.6185x over previous
import functools

import jax
import jax.numpy as jnp
from jax import lax
from jax.experimental import pallas as pl
from jax.experimental.pallas import tpu as pltpu
from jax.experimental.pallas import tpu_sc as plsc

N_MEM = 100000
N_SUB = 10000
N_EDGE = 320000
D = 128
H = D // 2
R = 16

NC = 2
NS = 16
NW = NC * NS

EB = 128
NROWS = N_EDGE // EB
N_SUBP = 10240
XPT = N_SUBP // NW
N_ACC = 10112
ZPT = N_ACC // NS


def _vmesh():
    return plsc.VectorSubcoreMesh(core_axis_name="c", subcore_axis_name="s")


@functools.partial(
    pl.kernel,
    mesh=_vmesh(),
    out_type=jax.ShapeDtypeStruct((NC, N_SUBP, H), jnp.float32),
    compiler_params=pltpu.CompilerParams(use_tc_tiling_on_sc=False),
    scratch_types=[
        pltpu.VMEM((EB,), jnp.int32),
        pltpu.VMEM((EB, D), jnp.float32),
        pltpu.SemaphoreType.DMA,
    ],
)
def _sc_gather_x(mem_h, nid_h, x_h, xid_v, rows_v, sem):
    c = lax.axis_index("c")
    s = lax.axis_index("s")
    wid = s * NC + c
    for off in (0, 128, 192):
        base = wid * XPT + off
        pltpu.sync_copy(nid_h.at[pl.ds(base, EB)], xid_v)
        pltpu.async_copy(mem_h.at[xid_v], rows_v, sem).wait()
        for hc in (0, 1):
            pltpu.sync_copy(rows_v.at[:, pl.ds(hc * H, H)],
                            x_h.at[hc, pl.ds(base, EB)])


@functools.partial(
    pl.kernel,
    mesh=_vmesh(),
    out_type=[
        jax.ShapeDtypeStruct((NC, N_ACC, H), jnp.float32),
        jax.ShapeDtypeStruct((N_ACC, R), jnp.float32),
        jax.ShapeDtypeStruct((N_ACC, R), jnp.float32),
    ],
    compiler_params=pltpu.CompilerParams(use_tc_tiling_on_sc=False),
    scratch_types=[
        pltpu.VMEM((1, EB), jnp.int32),
        pltpu.VMEM((1, EB), jnp.int32),
        pltpu.VMEM((EB, H), jnp.float32),
        pltpu.VMEM((EB, R), jnp.float32),
        pltpu.VMEM_SHARED((N_ACC, H), jnp.float32),
        pltpu.VMEM_SHARED((N_ACC, R), jnp.float32),
        pltpu.SemaphoreType.DMA,
    ],
)
def _sc_edge_agg(xh_h, src_h, dst_h, raw_h, z_d_h, z_r_h, ones_h,
                 xsum_h, rsum_h, cnt_h,
                 src_v, dst_v, rows_v, raw_v,
                 acc_sp, aux_sp, sem):
    c = lax.axis_index("c")
    s = lax.axis_index("s")

    zb = s * ZPT
    pltpu.sync_copy(z_d_h.at[pl.ds(zb, ZPT)], acc_sp.at[pl.ds(zb, ZPT)])
    pltpu.sync_copy(z_r_h.at[pl.ds(zb, ZPT)], aux_sp.at[pl.ds(zb, ZPT)])

    @pl.when(c == 1)
    def _():
        pltpu.sync_copy(ones_h, raw_v)

    plsc.subcore_barrier()

    nrows = lax.div(NROWS - s + NS - 1, NS)

    def body(i, carry):
        r = s + i * NS
        pltpu.sync_copy(src_h.at[r], src_v)
        pltpu.sync_copy(dst_h.at[r], dst_v)
        pltpu.async_copy(xh_h.at[c].at[src_v.at[0]], rows_v, sem).wait()

        @pl.when(c == 0)
        def _():
            pltpu.sync_copy(raw_h.at[pl.ds(r * EB, EB)], raw_v)

        pltpu.sync_copy(rows_v, acc_sp.at[dst_v.at[0]], add=True)
        pltpu.sync_copy(raw_v, aux_sp.at[dst_v.at[0]], add=True)
        return carry

    lax.fori_loop(0, nrows, body, 0)

    plsc.subcore_barrier()

    pltpu.sync_copy(acc_sp.at[pl.ds(zb, ZPT)], xsum_h.at[c, pl.ds(zb, ZPT)])

    @pl.when(c == 0)
    def _():
        pltpu.sync_copy(aux_sp.at[pl.ds(zb, ZPT)], rsum_h.at[pl.ds(zb, ZPT)])

    @pl.when(c == 1)
    def _():
        pltpu.sync_copy(aux_sp.at[pl.ds(zb, ZPT)], cnt_h.at[pl.ds(zb, ZPT)])


BM = 632


def _dense_body(x_ref, xs_ref, rs_ref, ct_ref, a0_ref, a1_ref, b_ref,
                wl0_ref, wl1_ref, wr_ref, bl_ref, blin_ref, br_ref, o_ref):
    ct = ct_ref[:, 0:1]
    num = (jnp.dot(xs_ref[0], a0_ref[...], preferred_element_type=jnp.float32)
           + jnp.dot(xs_ref[1], a1_ref[...], preferred_element_type=jnp.float32)
           + jnp.dot(rs_ref[...], b_ref[...], preferred_element_type=jnp.float32))
    mean = jnp.where(ct > 0.0, num / jnp.maximum(ct, 1.0) + blin_ref[...], 0.0)
    h = jnp.maximum(mean, 0.0)
    o = (jnp.dot(x_ref[0], wl0_ref[...], preferred_element_type=jnp.float32)
         + jnp.dot(x_ref[1], wl1_ref[...], preferred_element_type=jnp.float32)
         + bl_ref[...]
         + jnp.dot(h, wr_ref[...], preferred_element_type=jnp.float32)
         + br_ref[...])
    o_ref[...] = jnp.maximum(o, 0.0)


_dense = pl.pallas_call(
    _dense_body,
    grid=(N_ACC // BM,),
    in_specs=[
        pl.BlockSpec((NC, BM, H), lambda i: (0, i, 0)),
        pl.BlockSpec((NC, BM, H), lambda i: (0, i, 0)),
        pl.BlockSpec((BM, R), lambda i: (i, 0)),
        pl.BlockSpec((BM, R), lambda i: (i, 0)),
        pl.BlockSpec((H, D), lambda i: (0, 0)),
        pl.BlockSpec((H, D), lambda i: (0, 0)),
        pl.BlockSpec((R, D), lambda i: (0, 0)),
        pl.BlockSpec((H, D), lambda i: (0, 0)),
        pl.BlockSpec((H, D), lambda i: (0, 0)),
        pl.BlockSpec((D, D), lambda i: (0, 0)),
        pl.BlockSpec((1, D), lambda i: (0, 0)),
        pl.BlockSpec((1, D), lambda i: (0, 0)),
        pl.BlockSpec((1, D), lambda i: (0, 0)),
    ],
    out_specs=pl.BlockSpec((BM, D), lambda i: (i, 0)),
    out_shape=jax.ShapeDtypeStruct((N_ACC, D), jnp.float32),
)


def kernel(memory, raw_msg, W_lin, b_lin, W_l, b_l, W_r, b_r, n_id, edge_index):
    src = edge_index[0].reshape(NROWS, 1, EB)
    dst = edge_index[1].reshape(NROWS, 1, EB)
    nid_pad = jnp.pad(n_id, (0, N_SUBP - N_SUB))
    zeros_d = jnp.zeros((N_ACC, H), jnp.float32)
    zeros_r = jnp.zeros((N_ACC, R), jnp.float32)
    ones_r = jnp.ones((EB, R), jnp.float32)
    xh = _sc_gather_x(memory, nid_pad)
    xsum, rsum, cnt = _sc_edge_agg(xh, src, dst, raw_msg,
                                   zeros_d, zeros_r, ones_r)
    out = _dense(xh[:, :N_ACC], xsum, rsum, cnt,
                 W_lin[:H], W_lin[H:D], W_lin[D:], W_l[:H], W_l[H:],
                 W_r, b_l.reshape(1, D), b_lin.reshape(1, D),
                 b_r.reshape(1, D))
    return out[:N_SUB]


# speedup vs baseline: 7.4448x; 2.0574x over previous
import functools

import jax
import jax.numpy as jnp
from jax import lax
from jax.experimental import pallas as pl
from jax.experimental.pallas import tpu as pltpu
from jax.experimental.pallas import tpu_sc as plsc

N_MEM = 100000
N_SUB = 10000
N_EDGE = 320000
D = 128
H = D // 2
R = 16

NC = 2
NS = 16
NW = NC * NS

EB = 128
G = 2
GE = G * EB
NG = N_EDGE // GE
N_SUBP = 10240
XPT = N_SUBP // NW
N_ACC = 10112
ZPT = N_ACC // NS


def _vmesh():
    return plsc.VectorSubcoreMesh(core_axis_name="c", subcore_axis_name="s")


@functools.partial(
    pl.kernel,
    mesh=_vmesh(),
    out_type=jax.ShapeDtypeStruct((NC, N_SUBP, H), jnp.float32),
    compiler_params=pltpu.CompilerParams(use_tc_tiling_on_sc=False),
    scratch_types=[
        pltpu.VMEM((EB,), jnp.int32),
        pltpu.VMEM((EB, D), jnp.float32),
        pltpu.SemaphoreType.DMA,
    ],
)
def _sc_gather_x(mem_h, nid_h, x_h, xid_v, rows_v, sem):
    c = lax.axis_index("c")
    s = lax.axis_index("s")
    wid = s * NC + c
    for off in (0, 128, 192):
        base = wid * XPT + off
        pltpu.sync_copy(nid_h.at[pl.ds(base, EB)], xid_v)
        pltpu.async_copy(mem_h.at[xid_v], rows_v, sem).wait()
        for hc in (0, 1):
            pltpu.sync_copy(rows_v.at[:, pl.ds(hc * H, H)],
                            x_h.at[hc, pl.ds(base, EB)])


@functools.partial(
    pl.kernel,
    mesh=_vmesh(),
    out_type=[
        jax.ShapeDtypeStruct((NC, N_ACC, H), jnp.float32),
        jax.ShapeDtypeStruct((N_ACC, R), jnp.float32),
        jax.ShapeDtypeStruct((N_ACC, R), jnp.float32),
    ],
    compiler_params=pltpu.CompilerParams(use_tc_tiling_on_sc=False),
    scratch_types=[
        pltpu.VMEM((3, G, EB), jnp.int32),
        pltpu.VMEM((3, G, EB), jnp.int32),
        pltpu.VMEM((3, GE, R), jnp.float32),
        pltpu.VMEM((2, GE, H), jnp.float32),
        pltpu.VMEM((EB, R), jnp.float32),
        pltpu.VMEM_SHARED((N_ACC, H), jnp.float32),
        pltpu.VMEM_SHARED((N_ACC, R), jnp.float32),
        pltpu.SemaphoreType.DMA,
        pltpu.SemaphoreType.DMA,
        pltpu.SemaphoreType.DMA((3,)),
    ],
)
def _sc_edge_agg(xh_h, src_h, dst_h, raw_h, z_d_h, z_r_h, ones_h,
                 xsum_h, rsum_h, cnt_h,
                 srcb, dstb, rawb, rowsb, ones_v,
                 acc_sp, aux_sp, gsem, ssem, isems):
    c = lax.axis_index("c")
    s = lax.axis_index("s")

    zb = s * ZPT
    pltpu.sync_copy(z_d_h.at[pl.ds(zb, ZPT)], acc_sp.at[pl.ds(zb, ZPT)])
    pltpu.sync_copy(z_r_h.at[pl.ds(zb, ZPT)], aux_sp.at[pl.ds(zb, ZPT)])

    @pl.when(c == 1)
    def _():
        pltpu.sync_copy(ones_h, ones_v)

    pltpu.sync_copy(src_h.at[s], srcb.at[0])
    pltpu.sync_copy(dst_h.at[s], dstb.at[0])
    pltpu.async_copy(src_h.at[s + NS], srcb.at[1], isems.at[1])
    pltpu.async_copy(dst_h.at[s + NS], dstb.at[1], isems.at[1])

    @pl.when(c == 0)
    def _():
        pltpu.sync_copy(raw_h.at[s], rawb.at[0])
        pltpu.async_copy(raw_h.at[s + NS], rawb.at[1], isems.at[1])

    for j in range(G):
        pltpu.async_copy(xh_h.at[c].at[srcb.at[0, j]],
                         rowsb.at[0, pl.ds(j * EB, EB)], gsem)

    plsc.subcore_barrier()

    ng = lax.div(NG - s + NS - 1, NS)

    def body(i, carry):
        k0 = lax.rem(i, 3)
        k1 = lax.rem(i + 1, 3)
        k2 = lax.rem(i + 2, 3)
        p = lax.rem(i, 2)
        q = 1 - p
        g1 = s + (i + 1) * NS
        g2 = s + (i + 2) * NS

        for j in range(G):
            pltpu.make_async_copy(xh_h.at[c].at[srcb.at[k0, j]],
                                  rowsb.at[p, pl.ds(j * EB, EB)], gsem).wait()

        @pl.when(i + 2 < ng)
        def _():
            pltpu.async_copy(src_h.at[g2], srcb.at[k2], isems.at[k2])
            pltpu.async_copy(dst_h.at[g2], dstb.at[k2], isems.at[k2])

            @pl.when(c == 0)
            def _():
                pltpu.async_copy(raw_h.at[g2], rawb.at[k2], isems.at[k2])

        @pl.when(i + 1 < ng)
        def _():
            pltpu.make_async_copy(src_h.at[g1], srcb.at[k1], isems.at[k1]).wait()
            pltpu.make_async_copy(dst_h.at[g1], dstb.at[k1], isems.at[k1]).wait()

            @pl.when(c == 0)
            def _():
                pltpu.make_async_copy(raw_h.at[g1], rawb.at[k1],
                                      isems.at[k1]).wait()

            for j in range(G):
                pltpu.async_copy(xh_h.at[c].at[srcb.at[k1, j]],
                                 rowsb.at[q, pl.ds(j * EB, EB)], gsem)

        row_ds = [pltpu.async_copy(rowsb.at[p, pl.ds(j * EB, EB)],
                                   acc_sp.at[dstb.at[k0, j]], ssem, add=True)
                  for j in range(G)]

        @pl.when(c == 0)
        def _():
            for j in range(G):
                pltpu.async_copy(rawb.at[k0, pl.ds(j * EB, EB)],
                                 aux_sp.at[dstb.at[k0, j]], ssem, add=True)

        @pl.when(c == 1)
        def _():
            for j in range(G):
                pltpu.async_copy(ones_v, aux_sp.at[dstb.at[k0, j]], ssem,
                                 add=True)

        for d in row_ds:
            d.wait()
        for j in range(G):
            pltpu.make_async_copy(ones_v, aux_sp.at[dstb.at[k0, j]],
                                  ssem).wait()
        return carry

    lax.fori_loop(0, ng, body, 0)

    plsc.subcore_barrier()

    pltpu.sync_copy(acc_sp.at[pl.ds(zb, ZPT)], xsum_h.at[c, pl.ds(zb, ZPT)])

    @pl.when(c == 0)
    def _():
        pltpu.sync_copy(aux_sp.at[pl.ds(zb, ZPT)], rsum_h.at[pl.ds(zb, ZPT)])

    @pl.when(c == 1)
    def _():
        pltpu.sync_copy(aux_sp.at[pl.ds(zb, ZPT)], cnt_h.at[pl.ds(zb, ZPT)])


BM = 1000


def _dense_body(x_ref, xs_ref, rs_ref, ct_ref, a0_ref, a1_ref, b_ref,
                wl0_ref, wl1_ref, wr_ref, bl_ref, blin_ref, br_ref, o_ref):
    ct = ct_ref[:, 0:1]
    num = (jnp.dot(xs_ref[0], a0_ref[...], preferred_element_type=jnp.float32)
           + jnp.dot(xs_ref[1], a1_ref[...], preferred_element_type=jnp.float32)
           + jnp.dot(rs_ref[...], b_ref[...], preferred_element_type=jnp.float32))
    mean = jnp.where(ct > 0.0, num / jnp.maximum(ct, 1.0) + blin_ref[...], 0.0)
    h = jnp.maximum(mean, 0.0)
    o = (jnp.dot(x_ref[0], wl0_ref[...], preferred_element_type=jnp.float32)
         + jnp.dot(x_ref[1], wl1_ref[...], preferred_element_type=jnp.float32)
         + bl_ref[...]
         + jnp.dot(h, wr_ref[...], preferred_element_type=jnp.float32)
         + br_ref[...])
    o_ref[...] = jnp.maximum(o, 0.0)


_dense = pl.pallas_call(
    _dense_body,
    grid=(N_SUB // BM,),
    in_specs=[
        pl.BlockSpec((NC, BM, H), lambda i: (0, i, 0)),
        pl.BlockSpec((NC, BM, H), lambda i: (0, i, 0)),
        pl.BlockSpec((BM, R), lambda i: (i, 0)),
        pl.BlockSpec((BM, R), lambda i: (i, 0)),
        pl.BlockSpec((H, D), lambda i: (0, 0)),
        pl.BlockSpec((H, D), lambda i: (0, 0)),
        pl.BlockSpec((R, D), lambda i: (0, 0)),
        pl.BlockSpec((H, D), lambda i: (0, 0)),
        pl.BlockSpec((H, D), lambda i: (0, 0)),
        pl.BlockSpec((D, D), lambda i: (0, 0)),
        pl.BlockSpec((1, D), lambda i: (0, 0)),
        pl.BlockSpec((1, D), lambda i: (0, 0)),
        pl.BlockSpec((1, D), lambda i: (0, 0)),
    ],
    out_specs=pl.BlockSpec((BM, D), lambda i: (i, 0)),
    out_shape=jax.ShapeDtypeStruct((N_SUB, D), jnp.float32),
)


def kernel(memory, raw_msg, W_lin, b_lin, W_l, b_l, W_r, b_r, n_id, edge_index):
    src = edge_index[0].reshape(NG, G, EB)
    dst = edge_index[1].reshape(NG, G, EB)
    raw = raw_msg.reshape(NG, GE, R)
    nid_pad = jnp.pad(n_id, (0, N_SUBP - N_SUB))
    zeros_d = jnp.zeros((N_ACC, H), jnp.float32)
    zeros_r = jnp.zeros((N_ACC, R), jnp.float32)
    ones_r = jnp.ones((EB, R), jnp.float32)
    xh = _sc_gather_x(memory, nid_pad)
    xsum, rsum, cnt = _sc_edge_agg(xh, src, dst, raw,
                                   zeros_d, zeros_r, ones_r)
    return _dense(xh, xsum, rsum, cnt,
                  W_lin[:H], W_lin[H:D], W_lin[D:], W_l[:H], W_l[H:],
                  W_r, b_l.reshape(1, D), b_lin.reshape(1, D),
                  b_r.reshape(1, D))


# speedup vs baseline: 7.4625x; 1.0024x over previous
import functools

import jax
import jax.numpy as jnp
from jax import lax
from jax.experimental import pallas as pl
from jax.experimental.pallas import tpu as pltpu
from jax.experimental.pallas import tpu_sc as plsc

N_MEM = 100000
N_SUB = 10000
N_EDGE = 320000
D = 128
H = D // 2
R = 16

NC = 2
NS = 16
NW = NC * NS

EB = 128
G = 2
GE = G * EB
NG = N_EDGE // GE
N_SUBP = 10240
XPT = N_SUBP // NW
N_ACC = 10112
ZPT = N_ACC // NS


def _vmesh():
    return plsc.VectorSubcoreMesh(core_axis_name="c", subcore_axis_name="s")


@functools.partial(
    pl.kernel,
    mesh=_vmesh(),
    out_type=[
        jax.ShapeDtypeStruct((NC, N_SUBP, H), jnp.float32),
        jax.ShapeDtypeStruct((N_SUBP, D), jnp.float32),
    ],
    compiler_params=pltpu.CompilerParams(use_tc_tiling_on_sc=False),
    scratch_types=[
        pltpu.VMEM((EB,), jnp.int32),
        pltpu.VMEM((EB, D), jnp.float32),
        pltpu.SemaphoreType.DMA,
    ],
)
def _sc_gather_x(mem_h, nid_h, x_h, xf_h, xid_v, rows_v, sem):
    c = lax.axis_index("c")
    s = lax.axis_index("s")
    wid = s * NC + c
    for off in (0, 128, 192):
        base = wid * XPT + off
        pltpu.sync_copy(nid_h.at[pl.ds(base, EB)], xid_v)
        pltpu.async_copy(mem_h.at[xid_v], rows_v, sem).wait()
        pltpu.sync_copy(rows_v, xf_h.at[pl.ds(base, EB)])
        for hc in (0, 1):
            pltpu.sync_copy(rows_v.at[:, pl.ds(hc * H, H)],
                            x_h.at[hc, pl.ds(base, EB)])


@functools.partial(
    pl.kernel,
    mesh=_vmesh(),
    out_type=[
        jax.ShapeDtypeStruct((NC, N_ACC, H), jnp.float32),
        jax.ShapeDtypeStruct((N_ACC, R), jnp.float32),
        jax.ShapeDtypeStruct((N_ACC, R), jnp.float32),
    ],
    compiler_params=pltpu.CompilerParams(use_tc_tiling_on_sc=False),
    scratch_types=[
        pltpu.VMEM((3, G, EB), jnp.int32),
        pltpu.VMEM((3, G, EB), jnp.int32),
        pltpu.VMEM((3, GE, R), jnp.float32),
        pltpu.VMEM((2, GE, H), jnp.float32),
        pltpu.VMEM((EB, R), jnp.float32),
        pltpu.VMEM_SHARED((N_ACC, H), jnp.float32),
        pltpu.VMEM_SHARED((N_ACC, R), jnp.float32),
        pltpu.SemaphoreType.DMA,
        pltpu.SemaphoreType.DMA,
        pltpu.SemaphoreType.DMA((3,)),
    ],
)
def _sc_edge_agg(xh_h, src_h, dst_h, raw_h, z_d_h, z_r_h, ones_h,
                 xsum_h, rsum_h, cnt_h,
                 srcb, dstb, rawb, rowsb, ones_v,
                 acc_sp, aux_sp, gsem, ssem, isems):
    c = lax.axis_index("c")
    s = lax.axis_index("s")

    zb = s * ZPT
    pltpu.sync_copy(z_d_h.at[pl.ds(zb, ZPT)], acc_sp.at[pl.ds(zb, ZPT)])
    pltpu.sync_copy(z_r_h.at[pl.ds(zb, ZPT)], aux_sp.at[pl.ds(zb, ZPT)])

    @pl.when(c == 1)
    def _():
        pltpu.sync_copy(ones_h, ones_v)

    pltpu.sync_copy(src_h.at[s], srcb.at[0])
    pltpu.sync_copy(dst_h.at[s], dstb.at[0])
    pltpu.async_copy(src_h.at[s + NS], srcb.at[1], isems.at[1])
    pltpu.async_copy(dst_h.at[s + NS], dstb.at[1], isems.at[1])

    @pl.when(c == 0)
    def _():
        pltpu.sync_copy(raw_h.at[pl.ds(s * GE, GE)], rawb.at[0])
        pltpu.async_copy(raw_h.at[pl.ds((s + NS) * GE, GE)], rawb.at[1],
                         isems.at[1])

    for j in range(G):
        pltpu.async_copy(xh_h.at[c].at[srcb.at[0, j]],
                         rowsb.at[0, pl.ds(j * EB, EB)], gsem)

    plsc.subcore_barrier()

    ng = lax.div(NG - s + NS - 1, NS)

    def body(i, carry):
        k0 = lax.rem(i, 3)
        k1 = lax.rem(i + 1, 3)
        k2 = lax.rem(i + 2, 3)
        p = lax.rem(i, 2)
        q = 1 - p
        g1 = s + (i + 1) * NS
        g2 = s + (i + 2) * NS

        for j in range(G):
            pltpu.make_async_copy(xh_h.at[c].at[srcb.at[k0, j]],
                                  rowsb.at[p, pl.ds(j * EB, EB)], gsem).wait()

        @pl.when(i + 2 < ng)
        def _():
            pltpu.async_copy(src_h.at[g2], srcb.at[k2], isems.at[k2])
            pltpu.async_copy(dst_h.at[g2], dstb.at[k2], isems.at[k2])

            @pl.when(c == 0)
            def _():
                pltpu.async_copy(raw_h.at[pl.ds(g2 * GE, GE)], rawb.at[k2],
                                 isems.at[k2])

        @pl.when(i + 1 < ng)
        def _():
            pltpu.make_async_copy(src_h.at[g1], srcb.at[k1], isems.at[k1]).wait()
            pltpu.make_async_copy(dst_h.at[g1], dstb.at[k1], isems.at[k1]).wait()

            @pl.when(c == 0)
            def _():
                pltpu.make_async_copy(raw_h.at[pl.ds(g1 * GE, GE)],
                                      rawb.at[k1], isems.at[k1]).wait()

            for j in range(G):
                pltpu.async_copy(xh_h.at[c].at[srcb.at[k1, j]],
                                 rowsb.at[q, pl.ds(j * EB, EB)], gsem)

        row_ds = [pltpu.async_copy(rowsb.at[p, pl.ds(j * EB, EB)],
                                   acc_sp.at[dstb.at[k0, j]], ssem, add=True)
                  for j in range(G)]

        @pl.when(c == 0)
        def _():
            for j in range(G):
                pltpu.async_copy(rawb.at[k0, pl.ds(j * EB, EB)],
                                 aux_sp.at[dstb.at[k0, j]], ssem, add=True)

        @pl.when(c == 1)
        def _():
            for j in range(G):
                pltpu.async_copy(ones_v, aux_sp.at[dstb.at[k0, j]], ssem,
                                 add=True)

        for d in row_ds:
            d.wait()
        for j in range(G):
            pltpu.make_async_copy(ones_v, aux_sp.at[dstb.at[k0, j]],
                                  ssem).wait()
        return carry

    lax.fori_loop(0, ng, body, 0)

    plsc.subcore_barrier()

    pltpu.sync_copy(acc_sp.at[pl.ds(zb, ZPT)], xsum_h.at[c, pl.ds(zb, ZPT)])

    @pl.when(c == 0)
    def _():
        pltpu.sync_copy(aux_sp.at[pl.ds(zb, ZPT)], rsum_h.at[pl.ds(zb, ZPT)])

    @pl.when(c == 1)
    def _():
        pltpu.sync_copy(aux_sp.at[pl.ds(zb, ZPT)], cnt_h.at[pl.ds(zb, ZPT)])


BM = 1000


def _dense_body(x_ref, xs_ref, rs_ref, ct_ref, a0_ref, a1_ref, b_ref,
                wl_ref, wr_ref, bl_ref, blin_ref, br_ref, o_ref):
    ct = ct_ref[:, 0:1]
    num = (jnp.dot(xs_ref[0], a0_ref[...], preferred_element_type=jnp.float32)
           + jnp.dot(xs_ref[1], a1_ref[...], preferred_element_type=jnp.float32)
           + jnp.dot(rs_ref[...], b_ref[...], preferred_element_type=jnp.float32))
    mean = jnp.where(ct > 0.0, num / jnp.maximum(ct, 1.0) + blin_ref[...], 0.0)
    h = jnp.maximum(mean, 0.0)
    o = (jnp.dot(x_ref[...], wl_ref[...], preferred_element_type=jnp.float32)
         + bl_ref[...]
         + jnp.dot(h, wr_ref[...], preferred_element_type=jnp.float32)
         + br_ref[...])
    o_ref[...] = jnp.maximum(o, 0.0)


_dense = pl.pallas_call(
    _dense_body,
    grid=(N_SUB // BM,),
    in_specs=[
        pl.BlockSpec((BM, D), lambda i: (i, 0)),
        pl.BlockSpec((NC, BM, H), lambda i: (0, i, 0)),
        pl.BlockSpec((BM, R), lambda i: (i, 0)),
        pl.BlockSpec((BM, R), lambda i: (i, 0)),
        pl.BlockSpec((H, D), lambda i: (0, 0)),
        pl.BlockSpec((H, D), lambda i: (0, 0)),
        pl.BlockSpec((R, D), lambda i: (0, 0)),
        pl.BlockSpec((D, D), lambda i: (0, 0)),
        pl.BlockSpec((D, D), lambda i: (0, 0)),
        pl.BlockSpec((1, D), lambda i: (0, 0)),
        pl.BlockSpec((1, D), lambda i: (0, 0)),
        pl.BlockSpec((1, D), lambda i: (0, 0)),
    ],
    out_specs=pl.BlockSpec((BM, D), lambda i: (i, 0)),
    out_shape=jax.ShapeDtypeStruct((N_SUB, D), jnp.float32),
)


def kernel(memory, raw_msg, W_lin, b_lin, W_l, b_l, W_r, b_r, n_id, edge_index):
    src = edge_index[0].reshape(NG, G, EB)
    dst = edge_index[1].reshape(NG, G, EB)
    nid_pad = jnp.pad(n_id, (0, N_SUBP - N_SUB))
    zeros_d = jnp.zeros((N_ACC, H), jnp.float32)
    zeros_r = jnp.zeros((N_ACC, R), jnp.float32)
    ones_r = jnp.ones((EB, R), jnp.float32)
    xh, xf = _sc_gather_x(memory, nid_pad)
    xsum, rsum, cnt = _sc_edge_agg(xh, src, dst, raw_msg,
                                   zeros_d, zeros_r, ones_r)
    return _dense(xf, xsum, rsum, cnt,
                  W_lin[:H], W_lin[H:D], W_lin[D:], W_l,
                  W_r, b_l.reshape(1, D), b_lin.reshape(1, D),
                  b_r.reshape(1, D))


# speedup vs baseline: 7.6923x; 1.0308x over previous
import functools

import jax
import jax.numpy as jnp
from jax import lax
from jax.experimental import pallas as pl
from jax.experimental.pallas import tpu as pltpu
from jax.experimental.pallas import tpu_sc as plsc

N_MEM = 100000
N_SUB = 10000
N_EDGE = 320000
D = 128
H = D // 2
R = 16

NC = 2
NS = 16
NW = NC * NS

EB = 128
G = 4
GE = G * EB
NG = N_EDGE // GE
NGH = (NG + 1) // 2
N_SUBP = 10240
XPT = N_SUBP // NW
N_ACC = 10112
ZPT = N_ACC // NS


def _vmesh():
    return plsc.VectorSubcoreMesh(core_axis_name="c", subcore_axis_name="s")


@functools.partial(
    pl.kernel,
    mesh=_vmesh(),
    out_type=[
        jax.ShapeDtypeStruct((NC, N_SUBP, H), jnp.float32),
        jax.ShapeDtypeStruct((N_SUBP, D), jnp.float32),
    ],
    compiler_params=pltpu.CompilerParams(use_tc_tiling_on_sc=False),
    scratch_types=[
        pltpu.VMEM((EB,), jnp.int32),
        pltpu.VMEM((EB, D), jnp.float32),
        pltpu.SemaphoreType.DMA,
    ],
)
def _sc_gather_x(mem_h, nid_h, x_h, xf_h, xid_v, rows_v, sem):
    c = lax.axis_index("c")
    s = lax.axis_index("s")
    wid = s * NC + c
    for off in (0, 128, 192):
        base = wid * XPT + off
        pltpu.sync_copy(nid_h.at[pl.ds(base, EB)], xid_v)
        pltpu.async_copy(mem_h.at[xid_v], rows_v, sem).wait()
        pltpu.sync_copy(rows_v, xf_h.at[pl.ds(base, EB)])
        for hc in (0, 1):
            pltpu.sync_copy(rows_v.at[:, pl.ds(hc * H, H)],
                            x_h.at[hc, pl.ds(base, EB)])


@functools.partial(
    pl.kernel,
    mesh=_vmesh(),
    out_type=jax.ShapeDtypeStruct((NC, N_ACC, H), jnp.float32),
    compiler_params=pltpu.CompilerParams(use_tc_tiling_on_sc=False),
    scratch_types=[
        pltpu.VMEM((3, G, EB), jnp.int32),
        pltpu.VMEM((3, G, EB), jnp.int32),
        pltpu.VMEM((2, GE, H), jnp.float32),
        pltpu.VMEM_SHARED((N_ACC, H), jnp.float32),
        pltpu.SemaphoreType.DMA,
        pltpu.SemaphoreType.DMA,
        pltpu.SemaphoreType.DMA((3,)),
    ],
)
def _sc_edge_agg(xh_h, src_h, dst_h, z_d_h,
                 xsum_h,
                 srcb, dstb, rowsb,
                 acc_sp, gsem, ssem, isems):
    c = lax.axis_index("c")
    s = lax.axis_index("s")

    zb = s * ZPT
    pltpu.sync_copy(z_d_h.at[pl.ds(zb, ZPT)], acc_sp.at[pl.ds(zb, ZPT)])

    pltpu.sync_copy(src_h.at[s], srcb.at[0])
    pltpu.sync_copy(dst_h.at[s], dstb.at[0])
    pltpu.async_copy(src_h.at[s + NS], srcb.at[1], isems.at[1])
    pltpu.async_copy(dst_h.at[s + NS], dstb.at[1], isems.at[1])

    for j in range(G):
        pltpu.async_copy(xh_h.at[c].at[srcb.at[0, j]],
                         rowsb.at[0, pl.ds(j * EB, EB)], gsem)

    plsc.subcore_barrier()

    ng = lax.div(NG - s + NS - 1, NS)

    def body(i, carry):
        k0 = lax.rem(i, 3)
        k1 = lax.rem(i + 1, 3)
        k2 = lax.rem(i + 2, 3)
        p = lax.rem(i, 2)
        q = 1 - p
        g1 = s + (i + 1) * NS
        g2 = s + (i + 2) * NS

        for j in range(G):
            pltpu.make_async_copy(xh_h.at[c].at[srcb.at[k0, j]],
                                  rowsb.at[p, pl.ds(j * EB, EB)], gsem).wait()

        @pl.when(i + 2 < ng)
        def _():
            pltpu.async_copy(src_h.at[g2], srcb.at[k2], isems.at[k2])
            pltpu.async_copy(dst_h.at[g2], dstb.at[k2], isems.at[k2])

        @pl.when(i + 1 < ng)
        def _():
            pltpu.make_async_copy(src_h.at[g1], srcb.at[k1], isems.at[k1]).wait()
            pltpu.make_async_copy(dst_h.at[g1], dstb.at[k1], isems.at[k1]).wait()

            for j in range(G):
                pltpu.async_copy(xh_h.at[c].at[srcb.at[k1, j]],
                                 rowsb.at[q, pl.ds(j * EB, EB)], gsem)

        row_ds = [pltpu.async_copy(rowsb.at[p, pl.ds(j * EB, EB)],
                                   acc_sp.at[dstb.at[k0, j]], ssem, add=True)
                  for j in range(G)]
        for d in row_ds:
            d.wait()
        return carry

    lax.fori_loop(0, ng, body, 0)

    plsc.subcore_barrier()

    pltpu.sync_copy(acc_sp.at[pl.ds(zb, ZPT)], xsum_h.at[c, pl.ds(zb, ZPT)])


@functools.partial(
    pl.kernel,
    mesh=_vmesh(),
    out_type=[
        jax.ShapeDtypeStruct((NC, N_ACC, R), jnp.float32),
        jax.ShapeDtypeStruct((NC, N_ACC, R), jnp.float32),
    ],
    compiler_params=pltpu.CompilerParams(use_tc_tiling_on_sc=False),
    scratch_types=[
        pltpu.VMEM((3, G, EB), jnp.int32),
        pltpu.VMEM((3, GE, R), jnp.float32),
        pltpu.VMEM((EB, R), jnp.float32),
        pltpu.VMEM_SHARED((N_ACC, R), jnp.float32),
        pltpu.VMEM_SHARED((N_ACC, R), jnp.float32),
        pltpu.SemaphoreType.DMA,
        pltpu.SemaphoreType.DMA((3,)),
    ],
)
def _sc_aux_agg(dst_h, raw_h, z_r_h, ones_h,
                rsum_h, cnt_h,
                dstb, rawb, ones_v,
                rsum_sp, cnt_sp, ssem, isems):
    c = lax.axis_index("c")
    s = lax.axis_index("s")

    zb = s * ZPT
    pltpu.sync_copy(z_r_h.at[pl.ds(zb, ZPT)], rsum_sp.at[pl.ds(zb, ZPT)])
    pltpu.sync_copy(z_r_h.at[pl.ds(zb, ZPT)], cnt_sp.at[pl.ds(zb, ZPT)])
    pltpu.sync_copy(ones_h, ones_v)

    g_lo = c * NGH
    g_hi = jnp.minimum(NG, (c + 1) * NGH)

    g0 = g_lo + s
    pltpu.async_copy(dst_h.at[g0], dstb.at[0], isems.at[0])
    pltpu.async_copy(raw_h.at[pl.ds(g0 * GE, GE)], rawb.at[0], isems.at[0])
    pltpu.async_copy(dst_h.at[g0 + NS], dstb.at[1], isems.at[1])
    pltpu.async_copy(raw_h.at[pl.ds((g0 + NS) * GE, GE)], rawb.at[1],
                     isems.at[1])

    plsc.subcore_barrier()

    ng = lax.div(g_hi - g_lo - s + NS - 1, NS)

    def body(i, carry):
        k0 = lax.rem(i, 3)
        k2 = lax.rem(i + 2, 3)
        g = g0 + i * NS
        g2 = g0 + (i + 2) * NS

        pltpu.make_async_copy(dst_h.at[g], dstb.at[k0], isems.at[k0]).wait()
        pltpu.make_async_copy(raw_h.at[pl.ds(g * GE, GE)], rawb.at[k0],
                              isems.at[k0]).wait()

        @pl.when(i + 2 < ng)
        def _():
            pltpu.async_copy(dst_h.at[g2], dstb.at[k2], isems.at[k2])
            pltpu.async_copy(raw_h.at[pl.ds(g2 * GE, GE)], rawb.at[k2],
                             isems.at[k2])

        ds_ = []
        for j in range(G):
            ds_.append(pltpu.async_copy(rawb.at[k0, pl.ds(j * EB, EB)],
                                        rsum_sp.at[dstb.at[k0, j]], ssem,
                                        add=True))
            ds_.append(pltpu.async_copy(ones_v, cnt_sp.at[dstb.at[k0, j]],
                                        ssem, add=True))
        for d in ds_:
            d.wait()
        return carry

    lax.fori_loop(0, ng, body, 0)

    plsc.subcore_barrier()

    pltpu.sync_copy(rsum_sp.at[pl.ds(zb, ZPT)], rsum_h.at[c, pl.ds(zb, ZPT)])
    pltpu.sync_copy(cnt_sp.at[pl.ds(zb, ZPT)], cnt_h.at[c, pl.ds(zb, ZPT)])


BM = 1000


def _dense_body(x_ref, xs_ref, rs_ref, ct_ref, a0_ref, a1_ref, b_ref,
                wl_ref, wr_ref, bl_ref, blin_ref, br_ref, o_ref):
    ct = ct_ref[0, :, 0:1] + ct_ref[1, :, 0:1]
    rs = rs_ref[0] + rs_ref[1]
    num = (jnp.dot(xs_ref[0], a0_ref[...], preferred_element_type=jnp.float32)
           + jnp.dot(xs_ref[1], a1_ref[...], preferred_element_type=jnp.float32)
           + jnp.dot(rs, b_ref[...], preferred_element_type=jnp.float32))
    mean = jnp.where(ct > 0.0, num / jnp.maximum(ct, 1.0) + blin_ref[...], 0.0)
    h = jnp.maximum(mean, 0.0)
    o = (jnp.dot(x_ref[...], wl_ref[...], preferred_element_type=jnp.float32)
         + bl_ref[...]
         + jnp.dot(h, wr_ref[...], preferred_element_type=jnp.float32)
         + br_ref[...])
    o_ref[...] = jnp.maximum(o, 0.0)


_dense = pl.pallas_call(
    _dense_body,
    grid=(N_SUB // BM,),
    in_specs=[
        pl.BlockSpec((BM, D), lambda i: (i, 0)),
        pl.BlockSpec((NC, BM, H), lambda i: (0, i, 0)),
        pl.BlockSpec((NC, BM, R), lambda i: (0, i, 0)),
        pl.BlockSpec((NC, BM, R), lambda i: (0, i, 0)),
        pl.BlockSpec((H, D), lambda i: (0, 0)),
        pl.BlockSpec((H, D), lambda i: (0, 0)),
        pl.BlockSpec((R, D), lambda i: (0, 0)),
        pl.BlockSpec((D, D), lambda i: (0, 0)),
        pl.BlockSpec((D, D), lambda i: (0, 0)),
        pl.BlockSpec((1, D), lambda i: (0, 0)),
        pl.BlockSpec((1, D), lambda i: (0, 0)),
        pl.BlockSpec((1, D), lambda i: (0, 0)),
    ],
    out_specs=pl.BlockSpec((BM, D), lambda i: (i, 0)),
    out_shape=jax.ShapeDtypeStruct((N_SUB, D), jnp.float32),
)


def kernel(memory, raw_msg, W_lin, b_lin, W_l, b_l, W_r, b_r, n_id, edge_index):
    src = edge_index[0].reshape(NG, G, EB)
    dst = edge_index[1].reshape(NG, G, EB)
    nid_pad = jnp.pad(n_id, (0, N_SUBP - N_SUB))
    zeros_d = jnp.zeros((N_ACC, H), jnp.float32)
    zeros_r = jnp.zeros((N_ACC, R), jnp.float32)
    ones_r = jnp.ones((EB, R), jnp.float32)
    xh, xf = _sc_gather_x(memory, nid_pad)
    xsum = _sc_edge_agg(xh, src, dst, zeros_d)
    rsum, cnt = _sc_aux_agg(dst, raw_msg, zeros_r, ones_r)
    return _dense(xf, xsum, rsum, cnt,
                  W_lin[:H], W_lin[H:D], W_lin[D:], W_l,
                  W_r, b_l.reshape(1, D), b_lin.reshape(1, D),
                  b_r.reshape(1, D))


# speedup vs baseline: 9.6700x; 1.2571x over previous
import functools

import jax
import jax.numpy as jnp
from jax import lax
from jax.experimental import pallas as pl
from jax.experimental.pallas import tpu as pltpu
from jax.experimental.pallas import tpu_sc as plsc

N_MEM = 100000
N_SUB = 10000
N_EDGE = 320000
D = 128
H = D // 2
R = 16

NC = 2
NS = 16
NW = NC * NS

EB = 128
G = 4
GE = G * EB
NG = N_EDGE // GE
NGH = (NG + 1) // 2
N_SUBP = 10240
XPT = N_SUBP // NW
N_ACC = 10112
ZPT = N_ACC // NS


def _vmesh():
    return plsc.VectorSubcoreMesh(core_axis_name="c", subcore_axis_name="s")


@functools.partial(
    pl.kernel,
    mesh=_vmesh(),
    out_type=[
        jax.ShapeDtypeStruct((NC, N_SUBP, H), jnp.float32),
        jax.ShapeDtypeStruct((N_SUBP, D), jnp.float32),
    ],
    compiler_params=pltpu.CompilerParams(use_tc_tiling_on_sc=False),
    scratch_types=[
        pltpu.VMEM((EB,), jnp.int32),
        pltpu.VMEM((EB, D), jnp.float32),
        pltpu.SemaphoreType.DMA,
    ],
)
def _sc_gather_x(mem_h, nid_h, x_h, xf_h, xid_v, rows_v, sem):
    c = lax.axis_index("c")
    s = lax.axis_index("s")
    wid = s * NC + c
    for off in (0, 128, 192):
        base = wid * XPT + off
        pltpu.sync_copy(nid_h.at[pl.ds(base, EB)], xid_v)
        pltpu.async_copy(mem_h.at[xid_v], rows_v, sem).wait()
        pltpu.sync_copy(rows_v, xf_h.at[pl.ds(base, EB)])
        for hc in (0, 1):
            pltpu.sync_copy(rows_v.at[:, pl.ds(hc * H, H)],
                            x_h.at[hc, pl.ds(base, EB)])


@functools.partial(
    pl.kernel,
    mesh=_vmesh(),
    out_type=jax.ShapeDtypeStruct((NC, N_ACC, H), jnp.float32),
    compiler_params=pltpu.CompilerParams(use_tc_tiling_on_sc=False),
    scratch_types=[
        pltpu.VMEM((3, G, EB), jnp.int32),
        pltpu.VMEM((3, G, EB), jnp.int32),
        pltpu.VMEM((2, GE, H), jnp.float32),
        pltpu.VMEM_SHARED((N_ACC, H), jnp.float32),
        pltpu.SemaphoreType.DMA,
        pltpu.SemaphoreType.DMA,
        pltpu.SemaphoreType.DMA((3,)),
    ],
)
def _sc_edge_agg(xh_h, src_h, dst_h, z_d_h,
                 xsum_h,
                 srcb, dstb, rowsb,
                 acc_sp, gsem, ssem, isems):
    c = lax.axis_index("c")
    s = lax.axis_index("s")

    zb = s * ZPT
    pltpu.sync_copy(z_d_h.at[pl.ds(zb, ZPT)], acc_sp.at[pl.ds(zb, ZPT)])

    pltpu.sync_copy(src_h.at[s], srcb.at[0])
    pltpu.sync_copy(dst_h.at[s], dstb.at[0])
    pltpu.async_copy(src_h.at[s + NS], srcb.at[1], isems.at[1])
    pltpu.async_copy(dst_h.at[s + NS], dstb.at[1], isems.at[1])

    for j in range(G):
        pltpu.async_copy(xh_h.at[c].at[srcb.at[0, j]],
                         rowsb.at[0, pl.ds(j * EB, EB)], gsem)

    plsc.subcore_barrier()

    ng = lax.div(NG - s + NS - 1, NS)

    def body(i, carry):
        k0 = lax.rem(i, 3)
        k1 = lax.rem(i + 1, 3)
        k2 = lax.rem(i + 2, 3)
        p = lax.rem(i, 2)
        q = 1 - p
        g1 = s + (i + 1) * NS
        g2 = s + (i + 2) * NS

        for j in range(G):
            pltpu.make_async_copy(xh_h.at[c].at[srcb.at[k0, j]],
                                  rowsb.at[p, pl.ds(j * EB, EB)], gsem).wait()

        @pl.when(i + 2 < ng)
        def _():
            pltpu.async_copy(src_h.at[g2], srcb.at[k2], isems.at[k2])
            pltpu.async_copy(dst_h.at[g2], dstb.at[k2], isems.at[k2])

        @pl.when(i + 1 < ng)
        def _():
            pltpu.make_async_copy(src_h.at[g1], srcb.at[k1], isems.at[k1]).wait()
            pltpu.make_async_copy(dst_h.at[g1], dstb.at[k1], isems.at[k1]).wait()

            for j in range(G):
                pltpu.async_copy(xh_h.at[c].at[srcb.at[k1, j]],
                                 rowsb.at[q, pl.ds(j * EB, EB)], gsem)

        row_ds = [pltpu.async_copy(rowsb.at[p, pl.ds(j * EB, EB)],
                                   acc_sp.at[dstb.at[k0, j]], ssem, add=True)
                  for j in range(G)]
        for d in row_ds:
            d.wait()
        return carry

    lax.fori_loop(0, ng, body, 0)

    plsc.subcore_barrier()

    pltpu.sync_copy(acc_sp.at[pl.ds(zb, ZPT)], xsum_h.at[c, pl.ds(zb, ZPT)])


@functools.partial(
    pl.kernel,
    mesh=_vmesh(),
    out_type=[
        jax.ShapeDtypeStruct((NC, N_ACC, R), jnp.float32),
        jax.ShapeDtypeStruct((NC, N_ACC, R), jnp.float32),
    ],
    compiler_params=pltpu.CompilerParams(use_tc_tiling_on_sc=False),
    scratch_types=[
        pltpu.VMEM((3, G, EB), jnp.int32),
        pltpu.VMEM((3, GE, R), jnp.float32),
        pltpu.VMEM((EB, R), jnp.float32),
        pltpu.VMEM_SHARED((N_ACC, R), jnp.float32),
        pltpu.VMEM_SHARED((N_ACC, R), jnp.float32),
        pltpu.SemaphoreType.DMA,
        pltpu.SemaphoreType.DMA((3,)),
    ],
)
def _sc_aux_agg(dst_h, raw_h, z_r_h, ones_h, dep_h,
                rsum_h, cnt_h,
                dstb, rawb, ones_v,
                rsum_sp, cnt_sp, ssem, isems):
    del dep_h
    c = lax.axis_index("c")
    s = lax.axis_index("s")

    zb = s * ZPT
    pltpu.sync_copy(z_r_h.at[pl.ds(zb, ZPT)], rsum_sp.at[pl.ds(zb, ZPT)])
    pltpu.sync_copy(z_r_h.at[pl.ds(zb, ZPT)], cnt_sp.at[pl.ds(zb, ZPT)])
    pltpu.sync_copy(ones_h, ones_v)

    g_lo = c * NGH
    g_hi = jnp.minimum(NG, (c + 1) * NGH)

    g0 = g_lo + s
    pltpu.async_copy(dst_h.at[g0], dstb.at[0], isems.at[0])
    pltpu.async_copy(raw_h.at[pl.ds(g0 * GE, GE)], rawb.at[0], isems.at[0])
    pltpu.async_copy(dst_h.at[g0 + NS], dstb.at[1], isems.at[1])
    pltpu.async_copy(raw_h.at[pl.ds((g0 + NS) * GE, GE)], rawb.at[1],
                     isems.at[1])

    plsc.subcore_barrier()

    ng = lax.div(g_hi - g_lo - s + NS - 1, NS)

    def body(i, carry):
        k0 = lax.rem(i, 3)
        k2 = lax.rem(i + 2, 3)
        g = g0 + i * NS
        g2 = g0 + (i + 2) * NS

        pltpu.make_async_copy(dst_h.at[g], dstb.at[k0], isems.at[k0]).wait()
        pltpu.make_async_copy(raw_h.at[pl.ds(g * GE, GE)], rawb.at[k0],
                              isems.at[k0]).wait()

        @pl.when(i + 2 < ng)
        def _():
            pltpu.async_copy(dst_h.at[g2], dstb.at[k2], isems.at[k2])
            pltpu.async_copy(raw_h.at[pl.ds(g2 * GE, GE)], rawb.at[k2],
                             isems.at[k2])

        ds_ = []
        for j in range(G):
            ds_.append(pltpu.async_copy(rawb.at[k0, pl.ds(j * EB, EB)],
                                        rsum_sp.at[dstb.at[k0, j]], ssem,
                                        add=True))
            ds_.append(pltpu.async_copy(ones_v, cnt_sp.at[dstb.at[k0, j]],
                                        ssem, add=True))
        for d in ds_:
            d.wait()
        return carry

    lax.fori_loop(0, ng, body, 0)

    plsc.subcore_barrier()

    pltpu.sync_copy(rsum_sp.at[pl.ds(zb, ZPT)], rsum_h.at[c, pl.ds(zb, ZPT)])
    pltpu.sync_copy(cnt_sp.at[pl.ds(zb, ZPT)], cnt_h.at[c, pl.ds(zb, ZPT)])


BM = 1000


def _dense_body(x_ref, xs_ref, rs_ref, ct_ref, a0_ref, a1_ref, b_ref,
                wl_ref, wr_ref, bl_ref, blin_ref, br_ref, o_ref):
    ct = ct_ref[0, :, 0:1] + ct_ref[1, :, 0:1]
    rs = rs_ref[0] + rs_ref[1]
    num = (jnp.dot(xs_ref[0], a0_ref[...], preferred_element_type=jnp.float32)
           + jnp.dot(xs_ref[1], a1_ref[...], preferred_element_type=jnp.float32)
           + jnp.dot(rs, b_ref[...], preferred_element_type=jnp.float32))
    mean = jnp.where(ct > 0.0, num / jnp.maximum(ct, 1.0) + blin_ref[...], 0.0)
    h = jnp.maximum(mean, 0.0)
    o = (jnp.dot(x_ref[...], wl_ref[...], preferred_element_type=jnp.float32)
         + bl_ref[...]
         + jnp.dot(h, wr_ref[...], preferred_element_type=jnp.float32)
         + br_ref[...])
    o_ref[...] = jnp.maximum(o, 0.0)


_dense = pl.pallas_call(
    _dense_body,
    grid=(N_SUB // BM,),
    in_specs=[
        pl.BlockSpec((BM, D), lambda i: (i, 0)),
        pl.BlockSpec((NC, BM, H), lambda i: (0, i, 0)),
        pl.BlockSpec((NC, BM, R), lambda i: (0, i, 0)),
        pl.BlockSpec((NC, BM, R), lambda i: (0, i, 0)),
        pl.BlockSpec((H, D), lambda i: (0, 0)),
        pl.BlockSpec((H, D), lambda i: (0, 0)),
        pl.BlockSpec((R, D), lambda i: (0, 0)),
        pl.BlockSpec((D, D), lambda i: (0, 0)),
        pl.BlockSpec((D, D), lambda i: (0, 0)),
        pl.BlockSpec((1, D), lambda i: (0, 0)),
        pl.BlockSpec((1, D), lambda i: (0, 0)),
        pl.BlockSpec((1, D), lambda i: (0, 0)),
    ],
    out_specs=pl.BlockSpec((BM, D), lambda i: (i, 0)),
    out_shape=jax.ShapeDtypeStruct((N_SUB, D), jnp.float32),
)


def kernel(memory, raw_msg, W_lin, b_lin, W_l, b_l, W_r, b_r, n_id, edge_index):
    src = edge_index[0].reshape(NG, G, EB)
    dst = edge_index[1].reshape(NG, G, EB)
    nid_pad = jnp.pad(n_id, (0, N_SUBP - N_SUB))
    zeros_d = jnp.zeros((N_ACC, H), jnp.float32)
    zeros_r = jnp.zeros((N_ACC, R), jnp.float32)
    ones_r = jnp.ones((EB, R), jnp.float32)
    xh, xf = _sc_gather_x(memory, nid_pad)
    xsum = _sc_edge_agg(xh, src, dst, zeros_d)
    rsum, cnt = _sc_aux_agg(dst, raw_msg, zeros_r, ones_r, xsum)
    return _dense(xf, xsum, rsum, cnt,
                  W_lin[:H], W_lin[H:D], W_lin[D:], W_l,
                  W_r, b_l.reshape(1, D), b_lin.reshape(1, D),
                  b_r.reshape(1, D))


# speedup vs baseline: 9.8056x; 1.0140x over previous
import functools

import jax
import jax.numpy as jnp
from jax import lax
from jax.experimental import pallas as pl
from jax.experimental.pallas import tpu as pltpu
from jax.experimental.pallas import tpu_sc as plsc

N_MEM = 100000
N_SUB = 10000
N_EDGE = 320000
D = 128
H = D // 2
R = 16

NC = 2
NS = 16
NW = NC * NS

EB = 128
G = 4
GE = G * EB
NG = N_EDGE // GE
NGH = (NG + 1) // 2
N_SUBP = 10240
XPT = N_SUBP // NW
N_ACC = 10112
ZPT = N_ACC // NS


def _vmesh():
    return plsc.VectorSubcoreMesh(core_axis_name="c", subcore_axis_name="s")


@functools.partial(
    pl.kernel,
    mesh=_vmesh(),
    out_type=[
        jax.ShapeDtypeStruct((NC, N_SUBP, H), jnp.float32),
        jax.ShapeDtypeStruct((N_SUBP, D), jnp.float32),
    ],
    compiler_params=pltpu.CompilerParams(use_tc_tiling_on_sc=False),
    scratch_types=[
        pltpu.VMEM((2, EB), jnp.int32),
        pltpu.VMEM((2, EB, D), jnp.float32),
        pltpu.SemaphoreType.DMA,
        pltpu.SemaphoreType.DMA,
    ],
)
def _sc_gather_x(mem_h, nid_h, x_h, xf_h, xid_v, rows_v, gsem, wsem):
    c = lax.axis_index("c")
    s = lax.axis_index("s")
    wid = c * NS + s
    offs = (0, 128, 192)
    pltpu.sync_copy(nid_h.at[pl.ds(wid * XPT, EB)], xid_v.at[0])
    pltpu.async_copy(mem_h.at[xid_v.at[0]], rows_v.at[0], gsem)
    for j, off in enumerate(offs):
        base = wid * XPT + off
        p = j % 2
        pltpu.make_async_copy(mem_h.at[xid_v.at[p]], rows_v.at[p], gsem).wait()
        if j + 1 < len(offs):
            q = 1 - p
            base1 = wid * XPT + offs[j + 1]
            pltpu.sync_copy(nid_h.at[pl.ds(base1, EB)], xid_v.at[q])
            pltpu.async_copy(mem_h.at[xid_v.at[q]], rows_v.at[q], gsem)
        wds = [pltpu.async_copy(rows_v.at[p], xf_h.at[pl.ds(base, EB)], wsem)]
        for hc in (0, 1):
            wds.append(pltpu.async_copy(rows_v.at[p, :, pl.ds(hc * H, H)],
                                        x_h.at[hc, pl.ds(base, EB)], wsem))
        for d in wds:
            d.wait()


@functools.partial(
    pl.kernel,
    mesh=_vmesh(),
    out_type=jax.ShapeDtypeStruct((NC, N_ACC, H), jnp.float32),
    compiler_params=pltpu.CompilerParams(use_tc_tiling_on_sc=False),
    scratch_types=[
        pltpu.VMEM((3, G, EB), jnp.int32),
        pltpu.VMEM((3, G, EB), jnp.int32),
        pltpu.VMEM((2, GE, H), jnp.float32),
        pltpu.VMEM_SHARED((N_ACC, H), jnp.float32),
        pltpu.SemaphoreType.DMA,
        pltpu.SemaphoreType.DMA,
        pltpu.SemaphoreType.DMA((3,)),
    ],
)
def _sc_edge_agg(xh_h, src_h, dst_h, z_d_h,
                 xsum_h,
                 srcb, dstb, rowsb,
                 acc_sp, gsem, ssem, isems):
    c = lax.axis_index("c")
    s = lax.axis_index("s")

    zb = s * ZPT
    pltpu.sync_copy(z_d_h.at[pl.ds(zb, ZPT)], acc_sp.at[pl.ds(zb, ZPT)])

    pltpu.sync_copy(src_h.at[s], srcb.at[0])
    pltpu.sync_copy(dst_h.at[s], dstb.at[0])
    pltpu.async_copy(src_h.at[s + NS], srcb.at[1], isems.at[1])
    pltpu.async_copy(dst_h.at[s + NS], dstb.at[1], isems.at[1])

    for j in range(G):
        pltpu.async_copy(xh_h.at[c].at[srcb.at[0, j]],
                         rowsb.at[0, pl.ds(j * EB, EB)], gsem)

    plsc.subcore_barrier()

    ng = lax.div(NG - s + NS - 1, NS)

    def body(i, carry):
        k0 = lax.rem(i, 3)
        k1 = lax.rem(i + 1, 3)
        k2 = lax.rem(i + 2, 3)
        p = lax.rem(i, 2)
        q = 1 - p
        g1 = s + (i + 1) * NS
        g2 = s + (i + 2) * NS

        for j in range(G):
            pltpu.make_async_copy(xh_h.at[c].at[srcb.at[k0, j]],
                                  rowsb.at[p, pl.ds(j * EB, EB)], gsem).wait()

        @pl.when(i + 2 < ng)
        def _():
            pltpu.async_copy(src_h.at[g2], srcb.at[k2], isems.at[k2])
            pltpu.async_copy(dst_h.at[g2], dstb.at[k2], isems.at[k2])

        @pl.when(i + 1 < ng)
        def _():
            pltpu.make_async_copy(src_h.at[g1], srcb.at[k1], isems.at[k1]).wait()
            pltpu.make_async_copy(dst_h.at[g1], dstb.at[k1], isems.at[k1]).wait()

            for j in range(G):
                pltpu.async_copy(xh_h.at[c].at[srcb.at[k1, j]],
                                 rowsb.at[q, pl.ds(j * EB, EB)], gsem)

        row_ds = [pltpu.async_copy(rowsb.at[p, pl.ds(j * EB, EB)],
                                   acc_sp.at[dstb.at[k0, j]], ssem, add=True)
                  for j in range(G)]
        for d in row_ds:
            d.wait()
        return carry

    lax.fori_loop(0, ng, body, 0)

    plsc.subcore_barrier()

    pltpu.sync_copy(acc_sp.at[pl.ds(zb, ZPT)], xsum_h.at[c, pl.ds(zb, ZPT)])


@functools.partial(
    pl.kernel,
    mesh=_vmesh(),
    out_type=[
        jax.ShapeDtypeStruct((NC, N_ACC, R), jnp.float32),
        jax.ShapeDtypeStruct((NC, N_ACC, R), jnp.float32),
    ],
    compiler_params=pltpu.CompilerParams(use_tc_tiling_on_sc=False),
    scratch_types=[
        pltpu.VMEM((3, G, EB), jnp.int32),
        pltpu.VMEM((3, GE, R), jnp.float32),
        pltpu.VMEM((EB, R), jnp.float32),
        pltpu.VMEM_SHARED((N_ACC, R), jnp.float32),
        pltpu.VMEM_SHARED((N_ACC, R), jnp.float32),
        pltpu.SemaphoreType.DMA,
        pltpu.SemaphoreType.DMA((3,)),
    ],
)
def _sc_aux_agg(dst_h, raw_h, z_r_h, ones_h, dep_h,
                rsum_h, cnt_h,
                dstb, rawb, ones_v,
                rsum_sp, cnt_sp, ssem, isems):
    del dep_h
    c = lax.axis_index("c")
    s = lax.axis_index("s")

    zb = s * ZPT
    pltpu.sync_copy(z_r_h.at[pl.ds(zb, ZPT)], rsum_sp.at[pl.ds(zb, ZPT)])
    pltpu.sync_copy(z_r_h.at[pl.ds(zb, ZPT)], cnt_sp.at[pl.ds(zb, ZPT)])
    pltpu.sync_copy(ones_h, ones_v)

    g_lo = c * NGH
    g_hi = jnp.minimum(NG, (c + 1) * NGH)

    g0 = g_lo + s
    pltpu.async_copy(dst_h.at[g0], dstb.at[0], isems.at[0])
    pltpu.async_copy(raw_h.at[pl.ds(g0 * GE, GE)], rawb.at[0], isems.at[0])
    pltpu.async_copy(dst_h.at[g0 + NS], dstb.at[1], isems.at[1])
    pltpu.async_copy(raw_h.at[pl.ds((g0 + NS) * GE, GE)], rawb.at[1],
                     isems.at[1])

    plsc.subcore_barrier()

    ng = lax.div(g_hi - g_lo - s + NS - 1, NS)

    def body(i, carry):
        k0 = lax.rem(i, 3)
        k2 = lax.rem(i + 2, 3)
        g = g0 + i * NS
        g2 = g0 + (i + 2) * NS

        pltpu.make_async_copy(dst_h.at[g], dstb.at[k0], isems.at[k0]).wait()
        pltpu.make_async_copy(raw_h.at[pl.ds(g * GE, GE)], rawb.at[k0],
                              isems.at[k0]).wait()

        @pl.when(i + 2 < ng)
        def _():
            pltpu.async_copy(dst_h.at[g2], dstb.at[k2], isems.at[k2])
            pltpu.async_copy(raw_h.at[pl.ds(g2 * GE, GE)], rawb.at[k2],
                             isems.at[k2])

        ds_ = []
        for j in range(G):
            ds_.append(pltpu.async_copy(rawb.at[k0, pl.ds(j * EB, EB)],
                                        rsum_sp.at[dstb.at[k0, j]], ssem,
                                        add=True))
            ds_.append(pltpu.async_copy(ones_v, cnt_sp.at[dstb.at[k0, j]],
                                        ssem, add=True))
        for d in ds_:
            d.wait()
        return carry

    lax.fori_loop(0, ng, body, 0)

    plsc.subcore_barrier()

    pltpu.sync_copy(rsum_sp.at[pl.ds(zb, ZPT)], rsum_h.at[c, pl.ds(zb, ZPT)])
    pltpu.sync_copy(cnt_sp.at[pl.ds(zb, ZPT)], cnt_h.at[c, pl.ds(zb, ZPT)])


BM = 1000


def _dense_body(x_ref, xs_ref, rs_ref, ct_ref, a0_ref, a1_ref, b_ref,
                wl_ref, wr_ref, bl_ref, blin_ref, br_ref, o_ref):
    ct = ct_ref[0, :, 0:1] + ct_ref[1, :, 0:1]
    rs = rs_ref[0] + rs_ref[1]
    num = (jnp.dot(xs_ref[0], a0_ref[...], preferred_element_type=jnp.float32)
           + jnp.dot(xs_ref[1], a1_ref[...], preferred_element_type=jnp.float32)
           + jnp.dot(rs, b_ref[...], preferred_element_type=jnp.float32))
    mean = jnp.where(ct > 0.0, num / jnp.maximum(ct, 1.0) + blin_ref[...], 0.0)
    h = jnp.maximum(mean, 0.0)
    o = (jnp.dot(x_ref[...], wl_ref[...], preferred_element_type=jnp.float32)
         + bl_ref[...]
         + jnp.dot(h, wr_ref[...], preferred_element_type=jnp.float32)
         + br_ref[...])
    o_ref[...] = jnp.maximum(o, 0.0)


_dense = pl.pallas_call(
    _dense_body,
    grid=(N_SUB // BM,),
    in_specs=[
        pl.BlockSpec((BM, D), lambda i: (i, 0)),
        pl.BlockSpec((NC, BM, H), lambda i: (0, i, 0)),
        pl.BlockSpec((NC, BM, R), lambda i: (0, i, 0)),
        pl.BlockSpec((NC, BM, R), lambda i: (0, i, 0)),
        pl.BlockSpec((H, D), lambda i: (0, 0)),
        pl.BlockSpec((H, D), lambda i: (0, 0)),
        pl.BlockSpec((R, D), lambda i: (0, 0)),
        pl.BlockSpec((D, D), lambda i: (0, 0)),
        pl.BlockSpec((D, D), lambda i: (0, 0)),
        pl.BlockSpec((1, D), lambda i: (0, 0)),
        pl.BlockSpec((1, D), lambda i: (0, 0)),
        pl.BlockSpec((1, D), lambda i: (0, 0)),
    ],
    out_specs=pl.BlockSpec((BM, D), lambda i: (i, 0)),
    out_shape=jax.ShapeDtypeStruct((N_SUB, D), jnp.float32),
)


def kernel(memory, raw_msg, W_lin, b_lin, W_l, b_l, W_r, b_r, n_id, edge_index):
    src = edge_index[0].reshape(NG, G, EB)
    dst = edge_index[1].reshape(NG, G, EB)
    nid_pad = jnp.pad(n_id, (0, N_SUBP - N_SUB))
    zeros_d = jnp.zeros((N_ACC, H), jnp.float32)
    zeros_r = jnp.zeros((N_ACC, R), jnp.float32)
    ones_r = jnp.ones((EB, R), jnp.float32)
    xh, xf = _sc_gather_x(memory, nid_pad)
    xsum = _sc_edge_agg(xh, src, dst, zeros_d)
    rsum, cnt = _sc_aux_agg(dst, raw_msg, zeros_r, ones_r, xsum)
    return _dense(xf, xsum, rsum, cnt,
                  W_lin[:H], W_lin[H:D], W_lin[D:], W_l,
                  W_r, b_l.reshape(1, D), b_lin.reshape(1, D),
                  b_r.reshape(1, D))
